# baseline (speedup 1.0000x reference)
import functools

import jax
import jax.numpy as jnp
from jax import lax
from jax.experimental import pallas as pl
from jax.experimental.pallas import tpu as pltpu
from jax.experimental.pallas import tpu_sc as plsc

_NPAD = 3872
_DUMP = 3856


def _graph_scatter_body(idx_hbm, w_hbm, adj_hbm, idx_v, w_v, adj_sp, zero_v):
    cid = lax.axis_index("c")
    sid = lax.axis_index("s")

    @pl.when(jnp.logical_and(cid == 0, sid == 0))
    def _():
        pltpu.sync_copy(idx_hbm, idx_v)
        pltpu.sync_copy(w_hbm, w_v)
        z16 = jnp.zeros((16,), jnp.float32)
        for i in range(_NPAD // 16):
            zero_v[pl.ds(i * 16, 16)] = z16
        pltpu.sync_copy(zero_v, adj_sp)
        for j in range(8):
            sl = pl.ds(j * 128, 128)
            pltpu.sync_copy(w_v.at[sl], adj_sp.at[idx_v.at[sl]], add=True)
        pltpu.sync_copy(adj_sp, adj_hbm)


def _build_adj_sc(edge_index, edge_weight):
    E = edge_weight.shape[0]
    N = 62
    idx = edge_index[1] * N + edge_index[0]
    pad = 1024 - E
    idx_p = jnp.concatenate([idx, jnp.full((pad,), _DUMP, jnp.int32)])
    w_p = jnp.concatenate([edge_weight, jnp.zeros((pad,), jnp.float32)])
    mesh = plsc.VectorSubcoreMesh(core_axis_name="c", subcore_axis_name="s")
    adj_flat = pl.kernel(
        _graph_scatter_body,
        out_type=jax.ShapeDtypeStruct((_NPAD,), jnp.float32),
        mesh=mesh,
        scratch_types=[
            pltpu.VMEM((1024,), jnp.int32),
            pltpu.VMEM((1024,), jnp.float32),
            pltpu.VMEM_SHARED((_NPAD,), jnp.float32),
            pltpu.VMEM((_NPAD,), jnp.float32),
        ],
    )(idx_p, w_p)
    return adj_flat[:N * N].reshape(N, N)


def _dot_t(a, b):
    return lax.dot_general(a, b, (((1,), (1,)), ((), ())),
                           preferred_element_type=jnp.float32)


_MROW = 320


def _fused_body(x_ref, adj_ref, Wch_ref, gam_ref, bet_ref,
                Wih_ref, Whh_ref, bl_ref, g1_ref, b1_ref, Wlin_ref, blin_ref,
                out_ref, M_ref, s1_ref, s2_ref, Wf_ref, bf_ref,
                t1_ref, t2_ref, y2_ref, ch_ref, *, T, N, F, H, Bblk, Btot):
    NF = N * F
    phase = pl.program_id(0)
    blk = pl.program_id(1)
    eps = 1e-5
    f32 = jnp.float32

    @pl.when(jnp.logical_and(phase == 0, blk == 0))
    def _init():
        s1_ref[...] = jnp.zeros_like(s1_ref)
        s2_ref[...] = jnp.zeros_like(s2_ref)
        t1_ref[...] = jnp.zeros_like(t1_ref)
        t2_ref[...] = jnp.zeros_like(t2_ref)
        adj = adj_ref[...]
        deg = jnp.sum(adj, axis=0, keepdims=True)
        dinv = jnp.where(deg > 0, lax.rsqrt(deg), 0.0)
        outer = lax.dot_general(dinv, dinv, (((0,), (0,)), ((), ())),
                                preferred_element_type=f32)
        L = -(adj * outer)
        rN = lax.broadcasted_iota(jnp.int32, (N, N), 0)
        cN = lax.broadcasted_iota(jnp.int32, (N, N), 1)
        eyeN = (rN == cN).astype(f32)
        A2 = 2.0 * jnp.dot(L, L, preferred_element_type=f32) - eyeN
        rE = lax.broadcasted_iota(jnp.int32, (NF, N), 0)
        cE = lax.broadcasted_iota(jnp.int32, (NF, N), 1)
        EN = (rE // F == cE).astype(f32)
        rF = lax.broadcasted_iota(jnp.int32, (NF, F), 0)
        cF = lax.broadcasted_iota(jnp.int32, (NF, F), 1)
        EF = (rF % F == cF).astype(f32)
        Qs = []
        for A in (eyeN, L, A2):
            Qs.append(_dot_t(_dot_t(EN, A), EN))
        for t in range(T):
            acc = jnp.zeros((NF, NF), dtype=f32)
            for k in range(3):
                W = Wch_ref[t, k]
                R = _dot_t(lax.dot_general(
                    EF, W, (((1,), (0,)), ((), ())),
                    preferred_element_type=f32), EF)
                acc = acc + Qs[k] * R
            M_ref[pl.ds(t * _MROW, NF), :] = acc.astype(jnp.bfloat16)

    @pl.when(phase == 0)
    def _p0():
        xb = x_ref[...].astype(jnp.bfloat16)
        for t in range(T):
            ot = jnp.dot(xb[:, t, :], M_ref[pl.ds(t * _MROW, NF), :],
                         preferred_element_type=f32)
            ch_ref[pl.ds(t * Btot + blk * Bblk, Bblk), :] = ot.astype(jnp.bfloat16)
            s1_ref[pl.ds(t, 1), :] += jnp.sum(ot, axis=0, keepdims=True)
            s2_ref[pl.ds(t, 1), :] += jnp.sum(ot * ot, axis=0, keepdims=True)

    @pl.when(jnp.logical_and(phase == 1, blk == 0))
    def _finalize_bn():
        r5 = lax.broadcasted_iota(jnp.int32, (F, NF), 0)
        c5 = lax.broadcasted_iota(jnp.int32, (F, NF), 1) % F
        G2 = (r5 == c5).astype(f32)
        rG = lax.broadcasted_iota(jnp.int32, (NF, F), 0) % F
        cG = lax.broadcasted_iota(jnp.int32, (NF, F), 1)
        G = (rG == cG).astype(f32)
        cnt = float(Btot * N)
        fsum = jnp.dot(s1_ref[...], G, preferred_element_type=f32)
        fsq = jnp.dot(s2_ref[...], G, preferred_element_type=f32)
        mean = fsum[:T] / cnt
        var = fsq[:T] / cnt - mean * mean
        scale = gam_ref[...] * lax.rsqrt(var + eps)
        shift = bet_ref[...] - mean * scale
        scv = jnp.dot(scale, G2, preferred_element_type=f32)
        shv = jnp.dot(shift, G2, preferred_element_type=f32)
        Wih = Wih_ref[...]
        for t in range(T):
            Wf_ref[pl.ds(t * 4 * H, 4 * H), :] = (
                Wih * scv[t:t + 1, :]).astype(jnp.bfloat16)
            bf_ref[pl.ds(t, 1), :] = bl_ref[...] + _dot_t(shv[t:t + 1, :], Wih)

    @pl.when(phase == 1)
    def _p1():
        h = jnp.zeros((Bblk, H), dtype=f32)
        c = jnp.zeros((Bblk, H), dtype=f32)
        for t in range(T):
            ot = ch_ref[pl.ds(t * Btot + blk * Bblk, Bblk), :]
            gates = (_dot_t(ot, Wf_ref[pl.ds(t * 4 * H, 4 * H), :])
                     + _dot_t(h, Whh_ref[...])
                     + bf_ref[pl.ds(t, 1), :])
            i_g = jax.nn.sigmoid(gates[:, 0:H])
            f_g = jax.nn.sigmoid(gates[:, H:2 * H])
            g_g = jnp.tanh(gates[:, 2 * H:3 * H])
            o_g = jax.nn.sigmoid(gates[:, 3 * H:4 * H])
            c = f_g * c + i_g * g_g
            h = o_g * jnp.tanh(c)
            y2_ref[pl.ds(t * Btot + blk * Bblk, Bblk), :] = h
            t1_ref[pl.ds(t, 1), :] += jnp.sum(h, axis=0, keepdims=True)
            t2_ref[pl.ds(t, 1), :] += jnp.sum(h * h, axis=0, keepdims=True)

    @pl.when(phase == 2)
    def _p2():
        cnt2 = float(Btot * H)
        acc = jnp.zeros((Bblk, 3), dtype=f32)
        bias = blin_ref[...]
        for t in range(T):
            y2t = y2_ref[pl.ds(t * Btot + blk * Bblk, Bblk), :]
            m = jnp.sum(t1_ref[pl.ds(t, 1), :], axis=1, keepdims=True) / cnt2
            v = jnp.sum(t2_ref[pl.ds(t, 1), :], axis=1, keepdims=True) / cnt2 - m * m
            g1 = g1_ref[pl.ds(0, 1), pl.ds(t, 1)]
            b1 = b1_ref[pl.ds(0, 1), pl.ds(t, 1)]
            sc_t = g1 * lax.rsqrt(v + eps)
            sh_t = b1 - m * sc_t
            wl = Wlin_ref[:, pl.ds(t * H, H)]
            acc = acc + _dot_t(y2t, wl * sc_t)
            shv = jnp.zeros((1, H), dtype=f32) + sh_t
            bias = bias + _dot_t(shv, wl)
        out_ref[...] = acc + bias


def kernel(x, edge_index, edge_weight, W_cheb, b_cheb, bn_gamma, bn_beta,
           W_ih, W_hh, b_ih, b_hh, bn1_gamma, bn1_beta, W_lin, b_lin):
    B, T, N, F = x.shape
    NF = N * F
    H = W_hh.shape[1]
    del b_cheb

    adj = _build_adj_sc(edge_index, edge_weight)

    x_r = x.reshape(B, T, NF)
    bl = (b_ih + b_hh).reshape(1, 4 * H)
    blin = b_lin.reshape(1, 3)
    g1 = bn1_gamma.reshape(1, T)
    b1 = bn1_beta.reshape(1, T)

    Bblk = 512
    NB = B // Bblk

    body = functools.partial(_fused_body, T=T, N=N, F=F, H=H,
                             Bblk=Bblk, Btot=B)

    out = pl.pallas_call(
        body,
        grid=(3, NB),
        in_specs=[
            pl.BlockSpec((Bblk, T, NF),
                         lambda p, b: (jnp.where(p == 0, b, NB - 1), 0, 0)),
            pl.BlockSpec((N, N), lambda p, b: (0, 0)),
            pl.BlockSpec((T, 3, F, F), lambda p, b: (0, 0, 0, 0)),
            pl.BlockSpec((T, F), lambda p, b: (0, 0)),
            pl.BlockSpec((T, F), lambda p, b: (0, 0)),
            pl.BlockSpec((4 * H, NF), lambda p, b: (0, 0)),
            pl.BlockSpec((4 * H, H), lambda p, b: (0, 0)),
            pl.BlockSpec((1, 4 * H), lambda p, b: (0, 0)),
            pl.BlockSpec((1, T), lambda p, b: (0, 0)),
            pl.BlockSpec((1, T), lambda p, b: (0, 0)),
            pl.BlockSpec((3, T * H), lambda p, b: (0, 0)),
            pl.BlockSpec((1, 3), lambda p, b: (0, 0)),
        ],
        out_specs=pl.BlockSpec((Bblk, 3), lambda p, b: (b, 0)),
        out_shape=jax.ShapeDtypeStruct((B, 3), jnp.float32),
        scratch_shapes=[
            pltpu.VMEM((T * _MROW, NF), jnp.bfloat16),
            pltpu.VMEM((8, NF), jnp.float32),
            pltpu.VMEM((8, NF), jnp.float32),
            pltpu.VMEM((T * 4 * H, NF), jnp.bfloat16),
            pltpu.VMEM((8, 4 * H), jnp.float32),
            pltpu.VMEM((8, H), jnp.float32),
            pltpu.VMEM((8, H), jnp.float32),
            pltpu.VMEM((T * B, H), jnp.float32),
            pltpu.VMEM((T * B, NF), jnp.bfloat16),
        ],
        compiler_params=pltpu.CompilerParams(
            dimension_semantics=("arbitrary", "arbitrary"),
        ),
    )(x_r, adj, W_cheb, bn_gamma, bn_beta, W_ih, W_hh, bl, g1, b1,
      W_lin, blin)
    return out


# speedup vs baseline: 1.1162x; 1.1162x over previous
import functools

import jax
import jax.numpy as jnp
from jax import lax
from jax.experimental import pallas as pl
from jax.experimental.pallas import tpu as pltpu
from jax.experimental.pallas import tpu_sc as plsc

_NPAD = 3872
_DUMP = 3856


def _graph_scatter_body(idx_hbm, w_hbm, adj_hbm, idx_v, w_v, adj_sp, zero_v):
    cid = lax.axis_index("c")
    sid = lax.axis_index("s")

    @pl.when(jnp.logical_and(cid == 0, sid == 0))
    def _():
        pltpu.sync_copy(idx_hbm, idx_v)
        pltpu.sync_copy(w_hbm, w_v)
        z16 = jnp.zeros((16,), jnp.float32)
        for i in range(_NPAD // 16):
            zero_v[pl.ds(i * 16, 16)] = z16
        pltpu.sync_copy(zero_v, adj_sp)
        for j in range(8):
            sl = pl.ds(j * 128, 128)
            pltpu.sync_copy(w_v.at[sl], adj_sp.at[idx_v.at[sl]], add=True)
        pltpu.sync_copy(adj_sp, adj_hbm)


def _build_adj_sc(edge_index, edge_weight):
    E = edge_weight.shape[0]
    N = 62
    idx = edge_index[1] * N + edge_index[0]
    pad = 1024 - E
    idx_p = jnp.concatenate([idx, jnp.full((pad,), _DUMP, jnp.int32)])
    w_p = jnp.concatenate([edge_weight, jnp.zeros((pad,), jnp.float32)])
    mesh = plsc.VectorSubcoreMesh(core_axis_name="c", subcore_axis_name="s")
    adj_flat = pl.kernel(
        _graph_scatter_body,
        out_type=jax.ShapeDtypeStruct((_NPAD,), jnp.float32),
        mesh=mesh,
        scratch_types=[
            pltpu.VMEM((1024,), jnp.int32),
            pltpu.VMEM((1024,), jnp.float32),
            pltpu.VMEM_SHARED((_NPAD,), jnp.float32),
            pltpu.VMEM((_NPAD,), jnp.float32),
        ],
    )(idx_p, w_p)
    return adj_flat[:N * N].reshape(N, N)


def _dot_t(a, b):
    return lax.dot_general(a, b, (((1,), (1,)), ((), ())),
                           preferred_element_type=jnp.float32)


_MROW = 320


def _fused_body(x_ref, adj_ref, Wch_ref, gam_ref, bet_ref,
                Wih_ref, Whh_ref, bl_ref, g1_ref, b1_ref, Wlin_ref, blin_ref,
                out_ref, M_ref, s1_ref, s2_ref, Wf_ref, bf_ref,
                t1_ref, t2_ref, y2_ref, ch_ref, *, T, N, F, H, Bblk, Btot):
    NF = N * F
    phase = pl.program_id(0)
    blk = pl.program_id(1)
    eps = 1e-5
    f32 = jnp.float32

    @pl.when(jnp.logical_and(phase == 0, blk == 0))
    def _init():
        s1_ref[...] = jnp.zeros_like(s1_ref)
        s2_ref[...] = jnp.zeros_like(s2_ref)
        t1_ref[...] = jnp.zeros_like(t1_ref)
        t2_ref[...] = jnp.zeros_like(t2_ref)
        adj = adj_ref[...]
        deg = jnp.sum(adj, axis=0, keepdims=True)
        dinv = jnp.where(deg > 0, lax.rsqrt(deg), 0.0)
        outer = lax.dot_general(dinv, dinv, (((0,), (0,)), ((), ())),
                                preferred_element_type=f32)
        L = -(adj * outer)
        rN = lax.broadcasted_iota(jnp.int32, (N, N), 0)
        cN = lax.broadcasted_iota(jnp.int32, (N, N), 1)
        eyeN = (rN == cN).astype(f32)
        A2 = 2.0 * jnp.dot(L, L, preferred_element_type=f32) - eyeN
        rE = lax.broadcasted_iota(jnp.int32, (NF, N), 0)
        cE = lax.broadcasted_iota(jnp.int32, (NF, N), 1)
        EN = (rE // F == cE).astype(f32)
        rF = lax.broadcasted_iota(jnp.int32, (NF, F), 0)
        cF = lax.broadcasted_iota(jnp.int32, (NF, F), 1)
        EF = (rF % F == cF).astype(f32)
        Qs = []
        for A in (eyeN, L, A2):
            Qs.append(_dot_t(_dot_t(EN, A), EN))
        for t in range(T):
            acc = jnp.zeros((NF, NF), dtype=f32)
            for k in range(3):
                W = Wch_ref[t, k]
                R = _dot_t(lax.dot_general(
                    EF, W, (((1,), (0,)), ((), ())),
                    preferred_element_type=f32), EF)
                acc = acc + Qs[k] * R
            M_ref[pl.ds(t * _MROW, NF), :] = acc

    @pl.when(phase == 0)
    def _p0():
        for t in range(T):
            ot = jnp.dot(x_ref[:, t, :], M_ref[pl.ds(t * _MROW, NF), :],
                         preferred_element_type=f32)
            ch_ref[pl.ds(t * Btot + blk * Bblk, Bblk), :] = ot.astype(jnp.bfloat16)
            s1_ref[pl.ds(t, 1), :] += jnp.sum(ot, axis=0, keepdims=True)
            s2_ref[pl.ds(t, 1), :] += jnp.sum(ot * ot, axis=0, keepdims=True)

    @pl.when(jnp.logical_and(phase == 1, blk == 0))
    def _finalize_bn():
        r5 = lax.broadcasted_iota(jnp.int32, (F, NF), 0)
        c5 = lax.broadcasted_iota(jnp.int32, (F, NF), 1) % F
        G2 = (r5 == c5).astype(f32)
        rG = lax.broadcasted_iota(jnp.int32, (NF, F), 0) % F
        cG = lax.broadcasted_iota(jnp.int32, (NF, F), 1)
        G = (rG == cG).astype(f32)
        cnt = float(Btot * N)
        fsum = jnp.dot(s1_ref[...], G, preferred_element_type=f32)
        fsq = jnp.dot(s2_ref[...], G, preferred_element_type=f32)
        mean = fsum[:T] / cnt
        var = fsq[:T] / cnt - mean * mean
        scale = gam_ref[...] * lax.rsqrt(var + eps)
        shift = bet_ref[...] - mean * scale
        scv = jnp.dot(scale, G2, preferred_element_type=f32)
        shv = jnp.dot(shift, G2, preferred_element_type=f32)
        Wih = Wih_ref[...]
        for t in range(T):
            Wf_ref[pl.ds(t * 4 * H, 4 * H), :] = (
                Wih * scv[t:t + 1, :]).astype(jnp.bfloat16)
            bf_ref[pl.ds(t, 1), :] = bl_ref[...] + _dot_t(shv[t:t + 1, :], Wih)

    @pl.when(phase == 1)
    def _p1():
        h = jnp.zeros((Bblk, H), dtype=f32)
        c = jnp.zeros((Bblk, H), dtype=f32)
        for t in range(T):
            ot = ch_ref[pl.ds(t * Btot + blk * Bblk, Bblk), :]
            gates = (_dot_t(ot, Wf_ref[pl.ds(t * 4 * H, 4 * H), :])
                     + _dot_t(h, Whh_ref[...])
                     + bf_ref[pl.ds(t, 1), :])
            i_g = jax.nn.sigmoid(gates[:, 0:H])
            f_g = jax.nn.sigmoid(gates[:, H:2 * H])
            g_g = jnp.tanh(gates[:, 2 * H:3 * H])
            o_g = jax.nn.sigmoid(gates[:, 3 * H:4 * H])
            c = f_g * c + i_g * g_g
            h = o_g * jnp.tanh(c)
            y2_ref[pl.ds(t * Btot + blk * Bblk, Bblk), :] = h.astype(jnp.bfloat16)
            t1_ref[pl.ds(t, 1), :] += jnp.sum(h, axis=0, keepdims=True)
            t2_ref[pl.ds(t, 1), :] += jnp.sum(h * h, axis=0, keepdims=True)

    @pl.when(phase == 2)
    def _p2():
        cnt2 = float(Btot * H)
        acc = jnp.zeros((Bblk, 3), dtype=f32)
        bias = blin_ref[...]
        for t in range(T):
            y2t = y2_ref[pl.ds(t * Btot + blk * Bblk, Bblk), :]
            m = jnp.sum(t1_ref[pl.ds(t, 1), :], axis=1, keepdims=True) / cnt2
            v = jnp.sum(t2_ref[pl.ds(t, 1), :], axis=1, keepdims=True) / cnt2 - m * m
            g1 = g1_ref[pl.ds(0, 1), pl.ds(t, 1)]
            b1 = b1_ref[pl.ds(0, 1), pl.ds(t, 1)]
            sc_t = g1 * lax.rsqrt(v + eps)
            sh_t = b1 - m * sc_t
            wl = Wlin_ref[:, pl.ds(t * H, H)]
            acc = acc + _dot_t(y2t, (wl * sc_t).astype(jnp.bfloat16))
            shv = jnp.zeros((1, H), dtype=f32) + sh_t
            bias = bias + _dot_t(shv, wl)
        out_ref[...] = acc + bias


def kernel(x, edge_index, edge_weight, W_cheb, b_cheb, bn_gamma, bn_beta,
           W_ih, W_hh, b_ih, b_hh, bn1_gamma, bn1_beta, W_lin, b_lin):
    B, T, N, F = x.shape
    NF = N * F
    H = W_hh.shape[1]
    del b_cheb

    adj = _build_adj_sc(edge_index, edge_weight)

    x_r = x.reshape(B, T, NF)
    bl = (b_ih + b_hh).reshape(1, 4 * H)
    blin = b_lin.reshape(1, 3)
    g1 = bn1_gamma.reshape(1, T)
    b1 = bn1_beta.reshape(1, T)

    Bblk = 1024
    NB = B // Bblk

    body = functools.partial(_fused_body, T=T, N=N, F=F, H=H,
                             Bblk=Bblk, Btot=B)

    out = pl.pallas_call(
        body,
        grid=(3, NB),
        in_specs=[
            pl.BlockSpec((Bblk, T, NF),
                         lambda p, b: (jnp.where(p == 0, b, NB - 1), 0, 0)),
            pl.BlockSpec((N, N), lambda p, b: (0, 0)),
            pl.BlockSpec((T, 3, F, F), lambda p, b: (0, 0, 0, 0)),
            pl.BlockSpec((T, F), lambda p, b: (0, 0)),
            pl.BlockSpec((T, F), lambda p, b: (0, 0)),
            pl.BlockSpec((4 * H, NF), lambda p, b: (0, 0)),
            pl.BlockSpec((4 * H, H), lambda p, b: (0, 0)),
            pl.BlockSpec((1, 4 * H), lambda p, b: (0, 0)),
            pl.BlockSpec((1, T), lambda p, b: (0, 0)),
            pl.BlockSpec((1, T), lambda p, b: (0, 0)),
            pl.BlockSpec((3, T * H), lambda p, b: (0, 0)),
            pl.BlockSpec((1, 3), lambda p, b: (0, 0)),
        ],
        out_specs=pl.BlockSpec((Bblk, 3), lambda p, b: (b, 0)),
        out_shape=jax.ShapeDtypeStruct((B, 3), jnp.float32),
        scratch_shapes=[
            pltpu.VMEM((T * _MROW, NF), jnp.float32),
            pltpu.VMEM((8, NF), jnp.float32),
            pltpu.VMEM((8, NF), jnp.float32),
            pltpu.VMEM((T * 4 * H, NF), jnp.bfloat16),
            pltpu.VMEM((8, 4 * H), jnp.float32),
            pltpu.VMEM((8, H), jnp.float32),
            pltpu.VMEM((8, H), jnp.float32),
            pltpu.VMEM((T * B, H), jnp.bfloat16),
            pltpu.VMEM((T * B, NF), jnp.bfloat16),
        ],
        compiler_params=pltpu.CompilerParams(
            dimension_semantics=("arbitrary", "arbitrary"),
        ),
    )(x_r, adj, W_cheb, bn_gamma, bn_beta, W_ih, W_hh, bl, g1, b1,
      W_lin, blin)
    return out


# speedup vs baseline: 1.1242x; 1.0072x over previous
import functools

import jax
import jax.numpy as jnp
from jax import lax
from jax.experimental import pallas as pl
from jax.experimental.pallas import tpu as pltpu
from jax.experimental.pallas import tpu_sc as plsc

_NPAD = 64 * 128
_E = 992


def _graph_scatter_body(ei_hbm, ew_hbm, adj_hbm, row_v, col_v, idx_v, w_v,
                        adj_sp, zero_v):
    cid = lax.axis_index("c")
    sid = lax.axis_index("s")

    @pl.when(jnp.logical_and(cid == 0, sid == 0))
    def _():
        pltpu.sync_copy(ei_hbm.at[0], row_v)
        pltpu.sync_copy(ei_hbm.at[1], col_v)
        pltpu.sync_copy(ew_hbm, w_v)
        for i in range(_E // 16):
            sl = pl.ds(i * 16, 16)
            idx_v[sl] = col_v[sl] * 128 + row_v[sl]
        z16 = jnp.zeros((16,), jnp.float32)
        for i in range(_NPAD // 16):
            zero_v[pl.ds(i * 16, 16)] = z16
        pltpu.sync_copy(zero_v, adj_sp)
        for off, ln in [(0, 128), (128, 128), (256, 128), (384, 128),
                        (512, 128), (640, 128), (768, 128), (896, 96)]:
            sl = pl.ds(off, ln)
            pltpu.sync_copy(w_v.at[sl], adj_sp.at[idx_v.at[sl]], add=True)
        pltpu.sync_copy(adj_sp, adj_hbm)


def _build_adj_sc(edge_index, edge_weight):
    mesh = plsc.VectorSubcoreMesh(core_axis_name="c", subcore_axis_name="s")
    adj_flat = pl.kernel(
        _graph_scatter_body,
        out_type=jax.ShapeDtypeStruct((_NPAD,), jnp.float32),
        mesh=mesh,
        scratch_types=[
            pltpu.VMEM((_E,), jnp.int32),
            pltpu.VMEM((_E,), jnp.int32),
            pltpu.VMEM((_E,), jnp.int32),
            pltpu.VMEM((_E,), jnp.float32),
            pltpu.VMEM_SHARED((_NPAD,), jnp.float32),
            pltpu.VMEM((_NPAD,), jnp.float32),
        ],
    )(edge_index, edge_weight)
    return adj_flat.reshape(64, 128)


def _dot_t(a, b):
    return lax.dot_general(a, b, (((1,), (1,)), ((), ())),
                           preferred_element_type=jnp.float32)


_MROW = 320


def _fused_body(x_ref, adj_ref, Wch_ref, gam_ref, bet_ref,
                Wih_ref, Whh_ref, bl_ref, g1_ref, b1_ref, Wlin_ref, blin_ref,
                out_ref, M_ref, s1_ref, s2_ref, Wf_ref, bf_ref,
                t1_ref, t2_ref, y2_ref, ch_ref, *, T, N, F, H, Bblk, Btot):
    NF = N * F
    phase = pl.program_id(0)
    blk = pl.program_id(1)
    eps = 1e-5
    f32 = jnp.float32

    @pl.when(jnp.logical_and(phase == 0, blk == 0))
    def _init():
        s1_ref[...] = jnp.zeros_like(s1_ref)
        s2_ref[...] = jnp.zeros_like(s2_ref)
        t1_ref[...] = jnp.zeros_like(t1_ref)
        t2_ref[...] = jnp.zeros_like(t2_ref)
        adj = adj_ref[...][:N, :N]
        deg = jnp.sum(adj, axis=0, keepdims=True)
        dinv = jnp.where(deg > 0, lax.rsqrt(deg), 0.0)
        outer = lax.dot_general(dinv, dinv, (((0,), (0,)), ((), ())),
                                preferred_element_type=f32)
        L = -(adj * outer)
        rN = lax.broadcasted_iota(jnp.int32, (N, N), 0)
        cN = lax.broadcasted_iota(jnp.int32, (N, N), 1)
        eyeN = (rN == cN).astype(f32)
        A2 = 2.0 * jnp.dot(L, L, preferred_element_type=f32) - eyeN
        rE = lax.broadcasted_iota(jnp.int32, (NF, N), 0)
        cE = lax.broadcasted_iota(jnp.int32, (NF, N), 1)
        EN = (rE // F == cE).astype(f32)
        rF = lax.broadcasted_iota(jnp.int32, (NF, F), 0)
        cF = lax.broadcasted_iota(jnp.int32, (NF, F), 1)
        EF = (rF % F == cF).astype(f32)
        Qs = []
        for A in (eyeN, L, A2):
            Qs.append(_dot_t(_dot_t(EN, A), EN))
        for t in range(T):
            acc = jnp.zeros((NF, NF), dtype=f32)
            for k in range(3):
                W = Wch_ref[t, k]
                R = _dot_t(lax.dot_general(
                    EF, W, (((1,), (0,)), ((), ())),
                    preferred_element_type=f32), EF)
                acc = acc + Qs[k] * R
            M_ref[pl.ds(t * _MROW, NF), :] = acc

    @pl.when(phase == 0)
    def _p0():
        for t in range(T):
            ot = jnp.dot(x_ref[:, t, :], M_ref[pl.ds(t * _MROW, NF), :],
                         preferred_element_type=f32)
            ch_ref[pl.ds(t * Btot + blk * Bblk, Bblk), :] = ot.astype(jnp.bfloat16)
            s1_ref[pl.ds(t, 1), :] += jnp.sum(ot, axis=0, keepdims=True)
            s2_ref[pl.ds(t, 1), :] += jnp.sum(ot * ot, axis=0, keepdims=True)

    @pl.when(jnp.logical_and(phase == 1, blk == 0))
    def _finalize_bn():
        r5 = lax.broadcasted_iota(jnp.int32, (F, NF), 0)
        c5 = lax.broadcasted_iota(jnp.int32, (F, NF), 1) % F
        G2 = (r5 == c5).astype(f32)
        rG = lax.broadcasted_iota(jnp.int32, (NF, F), 0) % F
        cG = lax.broadcasted_iota(jnp.int32, (NF, F), 1)
        G = (rG == cG).astype(f32)
        cnt = float(Btot * N)
        fsum = jnp.dot(s1_ref[...], G, preferred_element_type=f32)
        fsq = jnp.dot(s2_ref[...], G, preferred_element_type=f32)
        mean = fsum[:T] / cnt
        var = fsq[:T] / cnt - mean * mean
        scale = gam_ref[...] * lax.rsqrt(var + eps)
        shift = bet_ref[...] - mean * scale
        scv = jnp.dot(scale, G2, preferred_element_type=f32)
        shv = jnp.dot(shift, G2, preferred_element_type=f32)
        Wih = Wih_ref[...]
        for t in range(T):
            Wf_ref[pl.ds(t * 4 * H, 4 * H), :] = (
                Wih * scv[t:t + 1, :]).astype(jnp.bfloat16)
            bf_ref[pl.ds(t, 1), :] = bl_ref[...] + _dot_t(shv[t:t + 1, :], Wih)

    @pl.when(phase == 1)
    def _p1():
        h = jnp.zeros((Bblk, H), dtype=f32)
        c = jnp.zeros((Bblk, H), dtype=f32)
        for t in range(T):
            ot = ch_ref[pl.ds(t * Btot + blk * Bblk, Bblk), :]
            gates = (_dot_t(ot, Wf_ref[pl.ds(t * 4 * H, 4 * H), :])
                     + _dot_t(h, Whh_ref[...])
                     + bf_ref[pl.ds(t, 1), :])
            i_g = jax.nn.sigmoid(gates[:, 0:H])
            f_g = jax.nn.sigmoid(gates[:, H:2 * H])
            g_g = jnp.tanh(gates[:, 2 * H:3 * H])
            o_g = jax.nn.sigmoid(gates[:, 3 * H:4 * H])
            c = f_g * c + i_g * g_g
            h = o_g * jnp.tanh(c)
            y2_ref[pl.ds(t * Btot + blk * Bblk, Bblk), :] = h.astype(jnp.bfloat16)
            t1_ref[pl.ds(t, 1), :] += jnp.sum(h, axis=0, keepdims=True)
            t2_ref[pl.ds(t, 1), :] += jnp.sum(h * h, axis=0, keepdims=True)

    @pl.when(phase == 2)
    def _p2():
        cnt2 = float(Btot * H)
        acc = jnp.zeros((Bblk, 3), dtype=f32)
        bias = blin_ref[...]
        for t in range(T):
            y2t = y2_ref[pl.ds(t * Btot + blk * Bblk, Bblk), :]
            m = jnp.sum(t1_ref[pl.ds(t, 1), :], axis=1, keepdims=True) / cnt2
            v = jnp.sum(t2_ref[pl.ds(t, 1), :], axis=1, keepdims=True) / cnt2 - m * m
            g1 = g1_ref[pl.ds(0, 1), pl.ds(t, 1)]
            b1 = b1_ref[pl.ds(0, 1), pl.ds(t, 1)]
            sc_t = g1 * lax.rsqrt(v + eps)
            sh_t = b1 - m * sc_t
            wl = Wlin_ref[:, pl.ds(t * H, H)]
            acc = acc + _dot_t(y2t, (wl * sc_t).astype(jnp.bfloat16))
            shv = jnp.zeros((1, H), dtype=f32) + sh_t
            bias = bias + _dot_t(shv, wl)
        out_ref[...] = acc + bias


def kernel(x, edge_index, edge_weight, W_cheb, b_cheb, bn_gamma, bn_beta,
           W_ih, W_hh, b_ih, b_hh, bn1_gamma, bn1_beta, W_lin, b_lin):
    B, T, N, F = x.shape
    NF = N * F
    H = W_hh.shape[1]
    del b_cheb

    adj = _build_adj_sc(edge_index, edge_weight)

    x_r = x.reshape(B, T, NF)
    bl = (b_ih + b_hh).reshape(1, 4 * H)
    blin = b_lin.reshape(1, 3)
    g1 = bn1_gamma.reshape(1, T)
    b1 = bn1_beta.reshape(1, T)

    Bblk = 1024
    NB = B // Bblk

    body = functools.partial(_fused_body, T=T, N=N, F=F, H=H,
                             Bblk=Bblk, Btot=B)

    out = pl.pallas_call(
        body,
        grid=(3, NB),
        in_specs=[
            pl.BlockSpec((Bblk, T, NF),
                         lambda p, b: (jnp.where(p == 0, b, NB - 1), 0, 0)),
            pl.BlockSpec((64, 128), lambda p, b: (0, 0)),
            pl.BlockSpec((T, 3, F, F), lambda p, b: (0, 0, 0, 0)),
            pl.BlockSpec((T, F), lambda p, b: (0, 0)),
            pl.BlockSpec((T, F), lambda p, b: (0, 0)),
            pl.BlockSpec((4 * H, NF), lambda p, b: (0, 0)),
            pl.BlockSpec((4 * H, H), lambda p, b: (0, 0)),
            pl.BlockSpec((1, 4 * H), lambda p, b: (0, 0)),
            pl.BlockSpec((1, T), lambda p, b: (0, 0)),
            pl.BlockSpec((1, T), lambda p, b: (0, 0)),
            pl.BlockSpec((3, T * H), lambda p, b: (0, 0)),
            pl.BlockSpec((1, 3), lambda p, b: (0, 0)),
        ],
        out_specs=pl.BlockSpec((Bblk, 3), lambda p, b: (b, 0)),
        out_shape=jax.ShapeDtypeStruct((B, 3), jnp.float32),
        scratch_shapes=[
            pltpu.VMEM((T * _MROW, NF), jnp.float32),
            pltpu.VMEM((8, NF), jnp.float32),
            pltpu.VMEM((8, NF), jnp.float32),
            pltpu.VMEM((T * 4 * H, NF), jnp.bfloat16),
            pltpu.VMEM((8, 4 * H), jnp.float32),
            pltpu.VMEM((8, H), jnp.float32),
            pltpu.VMEM((8, H), jnp.float32),
            pltpu.VMEM((T * B, H), jnp.bfloat16),
            pltpu.VMEM((T * B, NF), jnp.bfloat16),
        ],
        compiler_params=pltpu.CompilerParams(
            dimension_semantics=("arbitrary", "arbitrary"),
        ),
    )(x_r, adj, W_cheb, bn_gamma, bn_beta, W_ih, W_hh, bl, g1, b1,
      W_lin, blin)
    return out


# speedup vs baseline: 1.1244x; 1.0002x over previous
import functools

import jax
import jax.numpy as jnp
from jax import lax
from jax.experimental import pallas as pl
from jax.experimental.pallas import tpu as pltpu
from jax.experimental.pallas import tpu_sc as plsc

_NPAD = 64 * 128
_E = 992


def _graph_scatter_body(ei_hbm, ew_hbm, adj_hbm, row_v, col_v, idx_v, w_v,
                        adj_sp, zero_v):
    cid = lax.axis_index("c")
    sid = lax.axis_index("s")

    @pl.when(jnp.logical_and(cid == 0, sid == 0))
    def _():
        pltpu.sync_copy(ei_hbm.at[0], row_v)
        pltpu.sync_copy(ei_hbm.at[1], col_v)
        pltpu.sync_copy(ew_hbm, w_v)
        for i in range(_E // 16):
            sl = pl.ds(i * 16, 16)
            idx_v[sl] = col_v[sl] * 128 + row_v[sl]
        z16 = jnp.zeros((16,), jnp.float32)
        for i in range(_NPAD // 16):
            zero_v[pl.ds(i * 16, 16)] = z16
        pltpu.sync_copy(zero_v, adj_sp)
        for off, ln in [(0, 128), (128, 128), (256, 128), (384, 128),
                        (512, 128), (640, 128), (768, 128), (896, 96)]:
            sl = pl.ds(off, ln)
            pltpu.sync_copy(w_v.at[sl], adj_sp.at[idx_v.at[sl]], add=True)
        pltpu.sync_copy(adj_sp, adj_hbm)


def _build_adj_sc(edge_index, edge_weight):
    mesh = plsc.VectorSubcoreMesh(core_axis_name="c", subcore_axis_name="s")
    adj_flat = pl.kernel(
        _graph_scatter_body,
        out_type=jax.ShapeDtypeStruct((_NPAD,), jnp.float32),
        mesh=mesh,
        compiler_params=pltpu.CompilerParams(use_tc_tiling_on_sc=True),
        scratch_types=[
            pltpu.VMEM((_E,), jnp.int32),
            pltpu.VMEM((_E,), jnp.int32),
            pltpu.VMEM((_E,), jnp.int32),
            pltpu.VMEM((_E,), jnp.float32),
            pltpu.VMEM_SHARED((_NPAD,), jnp.float32),
            pltpu.VMEM((_NPAD,), jnp.float32),
        ],
    )(edge_index, edge_weight)
    return adj_flat.reshape(64, 128)


def _dot_t(a, b):
    return lax.dot_general(a, b, (((1,), (1,)), ((), ())),
                           preferred_element_type=jnp.float32)


_MROW = 320


def _fused_body(x_ref, adj_ref, Wch_ref, gam_ref, bet_ref,
                Wih_ref, Whh_ref, bl_ref, g1_ref, b1_ref, Wlin_ref, blin_ref,
                out_ref, M_ref, s1_ref, s2_ref, Wf_ref, bf_ref,
                t1_ref, t2_ref, y2_ref, ch_ref, *, T, N, F, H, Bblk, Btot):
    NF = N * F
    phase = pl.program_id(0)
    blk = pl.program_id(1)
    eps = 1e-5
    f32 = jnp.float32

    @pl.when(jnp.logical_and(phase == 0, blk == 0))
    def _init():
        s1_ref[...] = jnp.zeros_like(s1_ref)
        s2_ref[...] = jnp.zeros_like(s2_ref)
        t1_ref[...] = jnp.zeros_like(t1_ref)
        t2_ref[...] = jnp.zeros_like(t2_ref)
        adj = adj_ref[...][:N, :N]
        deg = jnp.sum(adj, axis=0, keepdims=True)
        dinv = jnp.where(deg > 0, lax.rsqrt(deg), 0.0)
        outer = lax.dot_general(dinv, dinv, (((0,), (0,)), ((), ())),
                                preferred_element_type=f32)
        L = -(adj * outer)
        rN = lax.broadcasted_iota(jnp.int32, (N, N), 0)
        cN = lax.broadcasted_iota(jnp.int32, (N, N), 1)
        eyeN = (rN == cN).astype(f32)
        A2 = 2.0 * jnp.dot(L, L, preferred_element_type=f32) - eyeN
        rE = lax.broadcasted_iota(jnp.int32, (NF, N), 0)
        cE = lax.broadcasted_iota(jnp.int32, (NF, N), 1)
        EN = (rE // F == cE).astype(f32)
        rF = lax.broadcasted_iota(jnp.int32, (NF, F), 0)
        cF = lax.broadcasted_iota(jnp.int32, (NF, F), 1)
        EF = (rF % F == cF).astype(f32)
        Qs = []
        for A in (eyeN, L, A2):
            Qs.append(_dot_t(_dot_t(EN, A), EN))
        for t in range(T):
            acc = jnp.zeros((NF, NF), dtype=f32)
            for k in range(3):
                W = Wch_ref[t, k]
                R = _dot_t(lax.dot_general(
                    EF, W, (((1,), (0,)), ((), ())),
                    preferred_element_type=f32), EF)
                acc = acc + Qs[k] * R
            M_ref[pl.ds(t * _MROW, NF), :] = acc

    @pl.when(phase == 0)
    def _p0():
        for t in range(T):
            ot = jnp.dot(x_ref[:, t, :], M_ref[pl.ds(t * _MROW, NF), :],
                         preferred_element_type=f32)
            ch_ref[pl.ds(t * Btot + blk * Bblk, Bblk), :] = ot.astype(jnp.bfloat16)
            s1_ref[pl.ds(t, 1), :] += jnp.sum(ot, axis=0, keepdims=True)
            s2_ref[pl.ds(t, 1), :] += jnp.sum(ot * ot, axis=0, keepdims=True)

    @pl.when(jnp.logical_and(phase == 1, blk == 0))
    def _finalize_bn():
        r5 = lax.broadcasted_iota(jnp.int32, (F, NF), 0)
        c5 = lax.broadcasted_iota(jnp.int32, (F, NF), 1) % F
        G2 = (r5 == c5).astype(f32)
        rG = lax.broadcasted_iota(jnp.int32, (NF, F), 0) % F
        cG = lax.broadcasted_iota(jnp.int32, (NF, F), 1)
        G = (rG == cG).astype(f32)
        cnt = float(Btot * N)
        fsum = jnp.dot(s1_ref[...], G, preferred_element_type=f32)
        fsq = jnp.dot(s2_ref[...], G, preferred_element_type=f32)
        mean = fsum[:T] / cnt
        var = fsq[:T] / cnt - mean * mean
        scale = gam_ref[...] * lax.rsqrt(var + eps)
        shift = bet_ref[...] - mean * scale
        scv = jnp.dot(scale, G2, preferred_element_type=f32)
        shv = jnp.dot(shift, G2, preferred_element_type=f32)
        Wih = Wih_ref[...]
        for t in range(T):
            Wf_ref[pl.ds(t * 4 * H, 4 * H), :] = (
                Wih * scv[t:t + 1, :]).astype(jnp.bfloat16)
            bf_ref[pl.ds(t, 1), :] = bl_ref[...] + _dot_t(shv[t:t + 1, :], Wih)

    @pl.when(phase == 1)
    def _p1():
        h = jnp.zeros((Bblk, H), dtype=f32)
        c = jnp.zeros((Bblk, H), dtype=f32)
        for t in range(T):
            ot = ch_ref[pl.ds(t * Btot + blk * Bblk, Bblk), :]
            gates = (_dot_t(ot, Wf_ref[pl.ds(t * 4 * H, 4 * H), :])
                     + _dot_t(h, Whh_ref[...])
                     + bf_ref[pl.ds(t, 1), :])
            i_g = jax.nn.sigmoid(gates[:, 0:H])
            f_g = jax.nn.sigmoid(gates[:, H:2 * H])
            g_g = jnp.tanh(gates[:, 2 * H:3 * H])
            o_g = jax.nn.sigmoid(gates[:, 3 * H:4 * H])
            c = f_g * c + i_g * g_g
            h = o_g * jnp.tanh(c)
            y2_ref[pl.ds(t * Btot + blk * Bblk, Bblk), :] = h.astype(jnp.bfloat16)
            t1_ref[pl.ds(t, 1), :] += jnp.sum(h, axis=0, keepdims=True)
            t2_ref[pl.ds(t, 1), :] += jnp.sum(h * h, axis=0, keepdims=True)

    @pl.when(phase == 2)
    def _p2():
        cnt2 = float(Btot * H)
        acc = jnp.zeros((Bblk, 3), dtype=f32)
        bias = blin_ref[...]
        for t in range(T):
            y2t = y2_ref[pl.ds(t * Btot + blk * Bblk, Bblk), :]
            m = jnp.sum(t1_ref[pl.ds(t, 1), :], axis=1, keepdims=True) / cnt2
            v = jnp.sum(t2_ref[pl.ds(t, 1), :], axis=1, keepdims=True) / cnt2 - m * m
            g1 = g1_ref[pl.ds(0, 1), pl.ds(t, 1)]
            b1 = b1_ref[pl.ds(0, 1), pl.ds(t, 1)]
            sc_t = g1 * lax.rsqrt(v + eps)
            sh_t = b1 - m * sc_t
            wl = Wlin_ref[:, pl.ds(t * H, H)]
            acc = acc + _dot_t(y2t, (wl * sc_t).astype(jnp.bfloat16))
            shv = jnp.zeros((1, H), dtype=f32) + sh_t
            bias = bias + _dot_t(shv, wl)
        out_ref[...] = acc + bias


def kernel(x, edge_index, edge_weight, W_cheb, b_cheb, bn_gamma, bn_beta,
           W_ih, W_hh, b_ih, b_hh, bn1_gamma, bn1_beta, W_lin, b_lin):
    B, T, N, F = x.shape
    NF = N * F
    H = W_hh.shape[1]
    del b_cheb

    adj = _build_adj_sc(edge_index, edge_weight)

    x_r = x.reshape(B, T, NF)
    bl = (b_ih + b_hh).reshape(1, 4 * H)
    blin = b_lin.reshape(1, 3)
    g1 = bn1_gamma.reshape(1, T)
    b1 = bn1_beta.reshape(1, T)

    Bblk = 1024
    NB = B // Bblk

    body = functools.partial(_fused_body, T=T, N=N, F=F, H=H,
                             Bblk=Bblk, Btot=B)

    out = pl.pallas_call(
        body,
        grid=(3, NB),
        in_specs=[
            pl.BlockSpec((Bblk, T, NF),
                         lambda p, b: (jnp.where(p == 0, b, NB - 1), 0, 0)),
            pl.BlockSpec((64, 128), lambda p, b: (0, 0)),
            pl.BlockSpec((T, 3, F, F), lambda p, b: (0, 0, 0, 0)),
            pl.BlockSpec((T, F), lambda p, b: (0, 0)),
            pl.BlockSpec((T, F), lambda p, b: (0, 0)),
            pl.BlockSpec((4 * H, NF), lambda p, b: (0, 0)),
            pl.BlockSpec((4 * H, H), lambda p, b: (0, 0)),
            pl.BlockSpec((1, 4 * H), lambda p, b: (0, 0)),
            pl.BlockSpec((1, T), lambda p, b: (0, 0)),
            pl.BlockSpec((1, T), lambda p, b: (0, 0)),
            pl.BlockSpec((3, T * H), lambda p, b: (0, 0)),
            pl.BlockSpec((1, 3), lambda p, b: (0, 0)),
        ],
        out_specs=pl.BlockSpec((Bblk, 3), lambda p, b: (b, 0)),
        out_shape=jax.ShapeDtypeStruct((B, 3), jnp.float32),
        scratch_shapes=[
            pltpu.VMEM((T * _MROW, NF), jnp.float32),
            pltpu.VMEM((8, NF), jnp.float32),
            pltpu.VMEM((8, NF), jnp.float32),
            pltpu.VMEM((T * 4 * H, NF), jnp.bfloat16),
            pltpu.VMEM((8, 4 * H), jnp.float32),
            pltpu.VMEM((8, H), jnp.float32),
            pltpu.VMEM((8, H), jnp.float32),
            pltpu.VMEM((T * B, H), jnp.bfloat16),
            pltpu.VMEM((T * B, NF), jnp.bfloat16),
        ],
        compiler_params=pltpu.CompilerParams(
            dimension_semantics=("arbitrary", "arbitrary"),
        ),
    )(x_r, adj, W_cheb, bn_gamma, bn_beta, W_ih, W_hh, bl, g1, b1,
      W_lin, blin)
    return out
